# Initial kernel scaffold; baseline (speedup 1.0000x reference)
import jax
import jax.numpy as jnp
from jax.experimental import pallas as pl


def kernel(feature, adjM_0, adjM_1, similary, feature_attr_0, feature_attr_1, W_gcn0, b_gcn0, W_gcn1, b_gcn1, W_gat, attn_l, attn_r, P1, pb1, P2, L1, lb1):
    raise NotImplementedError("write your pallas kernel here")


# speedup vs baseline: 21.0894x; 21.0894x over previous
import functools

import jax
import jax.numpy as jnp
from jax import lax
from jax.experimental import pallas as pl
from jax.experimental.pallas import tpu as pltpu
from jax.experimental.pallas import tpu_sc as plsc

_N = 4096
_E = 65536
_H = 128
_T = 128
_NC = 2
_NS = 16
_NW = _NC * _NS
_EW = _E // _NW
_ROWS_W = _N // _NS
_K = 128


def _proj_body(x_ref, w_ref, al_ref, ar_ref, feat_ref, el_ref, er_ref):
    f = jnp.dot(x_ref[...], w_ref[...], preferred_element_type=jnp.float32)
    feat_ref[...] = f
    el_ref[...] = jnp.sum(f * al_ref[...], axis=1, keepdims=True)
    er_ref[...] = jnp.sum(f * ar_ref[...], axis=1, keepdims=True)


def _gcn_body(adj0_ref, adj1_ref, fa0_ref, fa1_ref, w0_ref, b0_ref,
              w1_ref, b1_ref, l1a_ref, l1b_ref, lb1_ref, l_ref,
              s0_ref, s1_ref):
    @pl.when(pl.program_id(0) == 0)
    def _():
        s0_ref[...] = jnp.dot(fa0_ref[...], w0_ref[...],
                              preferred_element_type=jnp.float32)
        s1_ref[...] = jnp.dot(fa1_ref[...], w1_ref[...],
                              preferred_element_type=jnp.float32)

    h0 = jnp.tanh(jnp.dot(adj0_ref[...], s0_ref[...],
                          preferred_element_type=jnp.float32) + b0_ref[...])
    h1 = jnp.tanh(jnp.dot(adj1_ref[...], s1_ref[...],
                          preferred_element_type=jnp.float32) + b1_ref[...])
    l_ref[...] = jnp.tanh(
        jnp.dot(h0, l1a_ref[...], preferred_element_type=jnp.float32)
        + jnp.dot(h1, l1b_ref[...], preferred_element_type=jnp.float32)
        + lb1_ref[...])


def _sem_body(l_ref, a0_ref, a1_ref, p1_ref, pb1_ref, p2_ref, out_ref):
    lmat = l_ref[...]
    ff = jnp.tanh(a0_ref[...] + a1_ref[...])
    t0 = jnp.tanh(jnp.dot(lmat, p1_ref[...],
                          preferred_element_type=jnp.float32) + pb1_ref[...])
    t1 = jnp.tanh(jnp.dot(ff, p1_ref[...],
                          preferred_element_type=jnp.float32) + pb1_ref[...])
    w0 = jnp.sum(t0 * p2_ref[...]) / _N
    w1 = jnp.sum(t1 * p2_ref[...]) / _N
    m = jnp.maximum(w0, w1)
    e0 = jnp.exp(w0 - m)
    e1 = jnp.exp(w1 - m)
    den = e0 + e1
    out_ref[...] = (e0 / den) * lmat + (e1 / den) * ff


def _gat_a_body(src_hbm, dst_hbm, el_hbm, er_hbm, ex_hbm, esum_hbm,
                src_v, dst_v, el_v, er_v, ex_v, idx_v, dat_v, zb_v, esum_sh):
    c = lax.axis_index("c")
    s = lax.axis_index("s")
    wid = c * _NS + s
    base = wid * _EW

    for i in range(_ROWS_W // 16):
        zb_v[pl.ds(i * 16, 16)] = jnp.zeros((16,), jnp.float32)
    pltpu.sync_copy(zb_v, esum_sh.at[pl.ds(s * _ROWS_W, _ROWS_W)])

    pltpu.sync_copy(src_hbm.at[pl.ds(base, _EW)], src_v)
    pltpu.sync_copy(dst_hbm.at[pl.ds(base, _EW)], dst_v)
    pltpu.sync_copy(el_hbm, el_v)
    pltpu.sync_copy(er_hbm, er_v)

    def ch(i, carry):
        sl = pl.ds(i * 16, 16)
        elv = plsc.load_gather(el_v, [src_v[sl]])
        erv = plsc.load_gather(er_v, [dst_v[sl]])
        e = elv + erv
        e = jnp.where(e > 0.0, e, 0.2 * e)
        ex_v[sl] = jnp.exp(e)
        return carry
    lax.fori_loop(0, _EW // 16, ch, 0)

    pltpu.sync_copy(ex_v, ex_hbm.at[pl.ds(base, _EW)])
    plsc.subcore_barrier()

    def sc_ch(t, carry):
        for i in range(8):
            sl_l = pl.ds(i * 16, 16)
            sl_g = pl.ds(t * 128 + i * 16, 16)
            idx_v[sl_l] = dst_v[sl_g]
            dat_v[sl_l] = ex_v[sl_g]
        pltpu.sync_copy(dat_v, esum_sh.at[idx_v], add=True)
        return carry
    lax.fori_loop(0, _EW // 128, sc_ch, 0)

    plsc.subcore_barrier()
    pltpu.sync_copy(esum_sh.at[pl.ds(s * _ROWS_W, _ROWS_W)],
                    esum_hbm.at[c, pl.ds(s * _ROWS_W, _ROWS_W)])


def _gat_b_body(feat_hbm, src_hbm, dst_hbm, ex_hbm, esum_hbm, agg_hbm,
                src_v, dst_v, ex_v, es2_v, dinv_v, rows_v, idx_v, alp_v,
                zb_v, out_sh):
    c = lax.axis_index("c")
    s = lax.axis_index("s")
    wid = c * _NS + s
    base = wid * _EW

    def zrow(r, carry):
        for v in range(8):
            zb_v[r, pl.ds(v * 16, 16)] = jnp.zeros((16,), jnp.float32)
        return carry
    lax.fori_loop(0, 16, zrow, 0)

    def zcp(t, carry):
        pltpu.sync_copy(zb_v, out_sh.at[pl.ds(s * _ROWS_W + t * 16, 16)])
        return carry
    lax.fori_loop(0, _ROWS_W // 16, zcp, 0)

    pltpu.sync_copy(src_hbm.at[pl.ds(base, _EW)], src_v)
    pltpu.sync_copy(dst_hbm.at[pl.ds(base, _EW)], dst_v)
    pltpu.sync_copy(ex_hbm.at[pl.ds(base, _EW)], ex_v)
    pltpu.sync_copy(esum_hbm, es2_v)

    def dch(i, carry):
        sl = pl.ds(i * 16, 16)
        d = es2_v[0, sl] + es2_v[1, sl]
        dinv_v[sl] = 1.0 / (d + 1e-9)
        return carry
    lax.fori_loop(0, _N // 16, dch, 0)

    plsc.subcore_barrier()

    def chunk(t, carry):
        eb = t * _K
        for i in range(_K // 16):
            idx_v[pl.ds(i * 16, 16)] = src_v[pl.ds(eb + i * 16, 16)]
        pltpu.sync_copy(feat_hbm.at[idx_v], rows_v)
        for i in range(_K // 16):
            sl_l = pl.ds(i * 16, 16)
            sl_g = pl.ds(eb + i * 16, 16)
            dv = dst_v[sl_g]
            alp_v[sl_l] = ex_v[sl_g] * plsc.load_gather(dinv_v, [dv])

        def edge(j, carry2):
            av = plsc.load_gather(alp_v, [jnp.full((16,), j, jnp.int32)])
            for v in range(8):
                sl = pl.ds(v * 16, 16)
                rows_v[j, sl] = rows_v[j, sl] * av
            return carry2
        lax.fori_loop(0, _K, edge, 0)

        for i in range(_K // 16):
            idx_v[pl.ds(i * 16, 16)] = dst_v[pl.ds(eb + i * 16, 16)]
        pltpu.sync_copy(rows_v, out_sh.at[idx_v], add=True)
        return carry
    lax.fori_loop(0, _EW // _K, chunk, 0)

    plsc.subcore_barrier()
    pltpu.sync_copy(out_sh.at[pl.ds(s * _ROWS_W, _ROWS_W)],
                    agg_hbm.at[c, pl.ds(s * _ROWS_W, _ROWS_W)])


def kernel(feature, adjM_0, adjM_1, similary, feature_attr_0, feature_attr_1,
           W_gcn0, b_gcn0, W_gcn1, b_gcn1, W_gat, attn_l, attn_r,
           P1, pb1, P2, L1, lb1):
    f32 = jnp.float32

    feat, el, er = pl.pallas_call(
        _proj_body,
        out_shape=[jax.ShapeDtypeStruct((_N, _H), f32),
                   jax.ShapeDtypeStruct((_N, 1), f32),
                   jax.ShapeDtypeStruct((_N, 1), f32)],
    )(feature, W_gat, attn_l.reshape(1, _H), attn_r.reshape(1, _H))

    src = similary[0]
    dst = similary[1]
    el1 = el.reshape(_N)
    er1 = er.reshape(_N)

    mesh = plsc.VectorSubcoreMesh(core_axis_name="c", subcore_axis_name="s")
    sc_params = pltpu.CompilerParams(needs_layout_passes=False)
    gat_a = pl.kernel(
        _gat_a_body,
        out_type=[jax.ShapeDtypeStruct((_E,), f32),
                  jax.ShapeDtypeStruct((_NC, _N), f32)],
        mesh=mesh,
        compiler_params=sc_params,
        scratch_types=[
            pltpu.VMEM((_EW,), jnp.int32),
            pltpu.VMEM((_EW,), jnp.int32),
            pltpu.VMEM((_N,), f32),
            pltpu.VMEM((_N,), f32),
            pltpu.VMEM((_EW,), f32),
            pltpu.VMEM((128,), jnp.int32),
            pltpu.VMEM((128,), f32),
            pltpu.VMEM((_ROWS_W,), f32),
            pltpu.VMEM_SHARED((_N,), f32),
        ],
    )
    ex_all, esum_part = gat_a(src, dst, el1, er1)

    gat_b = pl.kernel(
        _gat_b_body,
        out_type=jax.ShapeDtypeStruct((_NC, _N, _H), f32),
        mesh=mesh,
        compiler_params=sc_params,
        scratch_types=[
            pltpu.VMEM((_EW,), jnp.int32),
            pltpu.VMEM((_EW,), jnp.int32),
            pltpu.VMEM((_EW,), f32),
            pltpu.VMEM((_NC, _N), f32),
            pltpu.VMEM((_N,), f32),
            pltpu.VMEM((_K, _H), f32),
            pltpu.VMEM((_K,), jnp.int32),
            pltpu.VMEM((_K,), f32),
            pltpu.VMEM((16, _H), f32),
            pltpu.VMEM_SHARED((_N, _H), f32),
        ],
    )
    agg_part = gat_b(feat, src, dst, ex_all, esum_part)

    br = 256
    full = lambda shape: pl.BlockSpec(shape, lambda i: (0,) * len(shape))
    l = pl.pallas_call(
        _gcn_body,
        grid=(_N // br,),
        in_specs=[
            pl.BlockSpec((br, _N), lambda i: (i, 0)),
            pl.BlockSpec((br, _N), lambda i: (i, 0)),
            full((_N, _T)), full((_N, _T)),
            full((_T, _H)), full((1, _H)),
            full((_T, _H)), full((1, _H)),
            full((_H, _H)), full((_H, _H)), full((1, _H)),
        ],
        out_specs=pl.BlockSpec((br, _H), lambda i: (i, 0)),
        out_shape=jax.ShapeDtypeStruct((_N, _H), f32),
        scratch_shapes=[pltpu.VMEM((_N, _H), f32), pltpu.VMEM((_N, _H), f32)],
    )(adjM_0, adjM_1, feature_attr_0, feature_attr_1,
      W_gcn0, b_gcn0.reshape(1, _H), W_gcn1, b_gcn1.reshape(1, _H),
      L1[:_H], L1[_H:], lb1.reshape(1, _H))

    out = pl.pallas_call(
        _sem_body,
        out_shape=jax.ShapeDtypeStruct((_N, _H), f32),
    )(l, agg_part[0], agg_part[1], P1, pb1.reshape(1, _H), P2.reshape(1, _H))
    return out


# speedup vs baseline: 25.9628x; 1.2311x over previous
import jax
import jax.numpy as jnp
from jax import lax
from jax.experimental import pallas as pl
from jax.experimental.pallas import tpu as pltpu
from jax.experimental.pallas import tpu_sc as plsc

_N = 4096
_E = 65536
_H = 128
_T = 128
_NC = 2
_NS = 16
_NW = _NC * _NS
_EW = _E // _NW
_ROWS_W = _N // _NS
_K = 128
_NT = _EW // _K


def _proj_body(x_ref, w_ref, al_ref, ar_ref, feat_ref, el_ref, er_ref):
    f = jnp.dot(x_ref[...], w_ref[...], preferred_element_type=jnp.float32)
    feat_ref[...] = f
    el_ref[...] = jnp.sum(f * al_ref[...], axis=1, keepdims=True)
    er_ref[...] = jnp.sum(f * ar_ref[...], axis=1, keepdims=True)


def _gcn_body(adj0_ref, adj1_ref, fa0_ref, fa1_ref, w0_ref, b0_ref,
              w1_ref, b1_ref, l1a_ref, l1b_ref, lb1_ref, l_ref,
              s0_ref, s1_ref):
    @pl.when(pl.program_id(0) == 0)
    def _():
        s0_ref[...] = jnp.dot(fa0_ref[...], w0_ref[...],
                              preferred_element_type=jnp.float32)
        s1_ref[...] = jnp.dot(fa1_ref[...], w1_ref[...],
                              preferred_element_type=jnp.float32)

    h0 = jnp.tanh(jnp.dot(adj0_ref[...], s0_ref[...],
                          preferred_element_type=jnp.float32) + b0_ref[...])
    h1 = jnp.tanh(jnp.dot(adj1_ref[...], s1_ref[...],
                          preferred_element_type=jnp.float32) + b1_ref[...])
    l_ref[...] = jnp.tanh(
        jnp.dot(h0, l1a_ref[...], preferred_element_type=jnp.float32)
        + jnp.dot(h1, l1b_ref[...], preferred_element_type=jnp.float32)
        + lb1_ref[...])


def _sem_body(l_ref, a0_ref, a1_ref, e0_ref, e1_ref, p1_ref, pb1_ref,
              p2_ref, out_ref):
    lmat = l_ref[...]
    dinv = 1.0 / (e0_ref[...] + e1_ref[...] + 1e-9)
    ff = jnp.tanh((a0_ref[...] + a1_ref[...]) * dinv)
    t0 = jnp.tanh(jnp.dot(lmat, p1_ref[...],
                          preferred_element_type=jnp.float32) + pb1_ref[...])
    t1 = jnp.tanh(jnp.dot(ff, p1_ref[...],
                          preferred_element_type=jnp.float32) + pb1_ref[...])
    w0 = jnp.sum(t0 * p2_ref[...]) / _N
    w1 = jnp.sum(t1 * p2_ref[...]) / _N
    m = jnp.maximum(w0, w1)
    e0 = jnp.exp(w0 - m)
    e1 = jnp.exp(w1 - m)
    den = e0 + e1
    out_ref[...] = (e0 / den) * lmat + (e1 / den) * ff


def _gat_sc_body(feat_hbm, src_hbm, dst_hbm, el_hbm, er_hbm,
                 esum_hbm, agg_hbm,
                 src_v, dst_v, el_v, er_v, ex_v, idx_v, dat_v,
                 zbe_v, zb_v,
                 gbuf0, gbuf1, sbuf0, sbuf1, ig0, ig1, is0, is1,
                 gsem0, gsem1, ssem0, ssem1,
                 esum_sh, out_sh):
    c = lax.axis_index("c")
    s = lax.axis_index("s")
    wid = c * _NS + s
    base = wid * _EW
    gbufs = (gbuf0, gbuf1)
    sbufs = (sbuf0, sbuf1)
    igs = (ig0, ig1)
    iss = (is0, is1)
    gsems = (gsem0, gsem1)
    ssems = (ssem0, ssem1)

    for i in range(_ROWS_W // 16):
        zbe_v[pl.ds(i * 16, 16)] = jnp.zeros((16,), jnp.float32)
    pltpu.sync_copy(zbe_v, esum_sh.at[pl.ds(s * _ROWS_W, _ROWS_W)])

    def zrow(r, carry):
        for v in range(8):
            zb_v[r, pl.ds(v * 16, 16)] = jnp.zeros((16,), jnp.float32)
        return carry
    lax.fori_loop(0, 16, zrow, 0)

    def zcp(t, carry):
        pltpu.sync_copy(zb_v, out_sh.at[pl.ds(s * _ROWS_W + t * 16, 16)])
        return carry
    lax.fori_loop(0, _ROWS_W // 16, zcp, 0)

    pltpu.sync_copy(src_hbm.at[pl.ds(base, _EW)], src_v)
    pltpu.sync_copy(dst_hbm.at[pl.ds(base, _EW)], dst_v)
    pltpu.sync_copy(el_hbm, el_v)
    pltpu.sync_copy(er_hbm, er_v)

    def ch(i, carry):
        sl = pl.ds(i * 16, 16)
        elv = plsc.load_gather(el_v, [src_v[sl]])
        erv = plsc.load_gather(er_v, [dst_v[sl]])
        e = elv + erv
        e = jnp.where(e > 0.0, e, 0.2 * e)
        ex_v[sl] = jnp.exp(e)
        return carry
    lax.fori_loop(0, _EW // 16, ch, 0)

    plsc.subcore_barrier()

    def fill_idx(buf, src_ref, t):
        for i in range(_K // 16):
            buf[pl.ds(i * 16, 16)] = src_ref[pl.ds(t * _K + i * 16, 16)]

    for b in range(2):
        fill_idx(igs[b], src_v, b)
        pltpu.async_copy(feat_hbm.at[igs[b]], gbufs[b], gsems[b])

    def sc_ch(t, carry):
        for i in range(8):
            sl_l = pl.ds(i * 16, 16)
            sl_g = pl.ds(t * 128 + i * 16, 16)
            idx_v[sl_l] = dst_v[sl_g]
            dat_v[sl_l] = ex_v[sl_g]
        pltpu.sync_copy(dat_v, esum_sh.at[idx_v], add=True)
        return carry
    lax.fori_loop(0, _EW // 128, sc_ch, 0)

    def pair(p, carry):
        for b in range(2):
            t = p * 2 + b
            pltpu.make_async_copy(feat_hbm.at[igs[b]], gbufs[b],
                                  gsems[b]).wait()

            @pl.when(t >= 2)
            def _():
                pltpu.make_async_copy(sbufs[b], out_sh.at[iss[b]],
                                      ssems[b]).wait()

            def edge(j, carry2):
                av = plsc.load_gather(
                    ex_v, [jnp.full((16,), t * _K + j, jnp.int32)])
                for v in range(8):
                    sl = pl.ds(v * 16, 16)
                    sbufs[b][j, sl] = gbufs[b][j, sl] * av
                return carry2
            lax.fori_loop(0, _K, edge, 0)

            fill_idx(iss[b], dst_v, t)
            pltpu.async_copy(sbufs[b], out_sh.at[iss[b]], ssems[b], add=True)

            @pl.when(t + 2 < _NT)
            def _():
                fill_idx(igs[b], src_v, t + 2)
                pltpu.async_copy(feat_hbm.at[igs[b]], gbufs[b], gsems[b])
        return carry
    lax.fori_loop(0, _NT // 2, pair, 0)

    for b in range(2):
        pltpu.make_async_copy(sbufs[b], out_sh.at[iss[b]], ssems[b]).wait()

    plsc.subcore_barrier()
    pltpu.sync_copy(esum_sh.at[pl.ds(s * _ROWS_W, _ROWS_W)],
                    esum_hbm.at[c, pl.ds(s * _ROWS_W, _ROWS_W)])
    pltpu.sync_copy(out_sh.at[pl.ds(s * _ROWS_W, _ROWS_W)],
                    agg_hbm.at[c, pl.ds(s * _ROWS_W, _ROWS_W)])


def kernel(feature, adjM_0, adjM_1, similary, feature_attr_0, feature_attr_1,
           W_gcn0, b_gcn0, W_gcn1, b_gcn1, W_gat, attn_l, attn_r,
           P1, pb1, P2, L1, lb1):
    f32 = jnp.float32

    feat, el, er = pl.pallas_call(
        _proj_body,
        out_shape=[jax.ShapeDtypeStruct((_N, _H), f32),
                   jax.ShapeDtypeStruct((_N, 1), f32),
                   jax.ShapeDtypeStruct((_N, 1), f32)],
    )(feature, W_gat, attn_l.reshape(1, _H), attn_r.reshape(1, _H))

    src = similary[0]
    dst = similary[1]
    el1 = el.reshape(_N)
    er1 = er.reshape(_N)

    mesh = plsc.VectorSubcoreMesh(core_axis_name="c", subcore_axis_name="s")
    sc_params = pltpu.CompilerParams(needs_layout_passes=False)
    gat_sc = pl.kernel(
        _gat_sc_body,
        out_type=[jax.ShapeDtypeStruct((_NC, _N), f32),
                  jax.ShapeDtypeStruct((_NC, _N, _H), f32)],
        mesh=mesh,
        compiler_params=sc_params,
        scratch_types=[
            pltpu.VMEM((_EW,), jnp.int32),
            pltpu.VMEM((_EW,), jnp.int32),
            pltpu.VMEM((_N,), f32),
            pltpu.VMEM((_N,), f32),
            pltpu.VMEM((_EW,), f32),
            pltpu.VMEM((128,), jnp.int32),
            pltpu.VMEM((128,), f32),
            pltpu.VMEM((_ROWS_W,), f32),
            pltpu.VMEM((16, _H), f32),
            pltpu.VMEM((_K, _H), f32),
            pltpu.VMEM((_K, _H), f32),
            pltpu.VMEM((_K, _H), f32),
            pltpu.VMEM((_K, _H), f32),
            pltpu.VMEM((_K,), jnp.int32),
            pltpu.VMEM((_K,), jnp.int32),
            pltpu.VMEM((_K,), jnp.int32),
            pltpu.VMEM((_K,), jnp.int32),
            pltpu.SemaphoreType.DMA,
            pltpu.SemaphoreType.DMA,
            pltpu.SemaphoreType.DMA,
            pltpu.SemaphoreType.DMA,
            pltpu.VMEM_SHARED((_N,), f32),
            pltpu.VMEM_SHARED((_N, _H), f32),
        ],
    )
    esum_part, agg_part = gat_sc(feat, src, dst, el1, er1)

    br = 256
    full = lambda shape: pl.BlockSpec(shape, lambda i: (0,) * len(shape))
    l = pl.pallas_call(
        _gcn_body,
        grid=(_N // br,),
        in_specs=[
            pl.BlockSpec((br, _N), lambda i: (i, 0)),
            pl.BlockSpec((br, _N), lambda i: (i, 0)),
            full((_N, _T)), full((_N, _T)),
            full((_T, _H)), full((1, _H)),
            full((_T, _H)), full((1, _H)),
            full((_H, _H)), full((_H, _H)), full((1, _H)),
        ],
        out_specs=pl.BlockSpec((br, _H), lambda i: (i, 0)),
        out_shape=jax.ShapeDtypeStruct((_N, _H), f32),
        scratch_shapes=[pltpu.VMEM((_N, _H), f32), pltpu.VMEM((_N, _H), f32)],
    )(adjM_0, adjM_1, feature_attr_0, feature_attr_1,
      W_gcn0, b_gcn0.reshape(1, _H), W_gcn1, b_gcn1.reshape(1, _H),
      L1[:_H], L1[_H:], lb1.reshape(1, _H))

    esum_col = esum_part.reshape(_NC, _N, 1)
    out = pl.pallas_call(
        _sem_body,
        out_shape=jax.ShapeDtypeStruct((_N, _H), f32),
    )(l, agg_part[0], agg_part[1], esum_col[0], esum_col[1],
      P1, pb1.reshape(1, _H), P2.reshape(1, _H))
    return out


# speedup vs baseline: 28.2965x; 1.0899x over previous
import jax
import jax.numpy as jnp
from jax import lax
from jax.experimental import pallas as pl
from jax.experimental.pallas import tpu as pltpu
from jax.experimental.pallas import tpu_sc as plsc

_N = 4096
_E = 65536
_H = 128
_T = 128
_NC = 2
_NS = 16
_NW = _NC * _NS
_EW = _E // _NW
_ROWS_W = _N // _NS
_K = 128
_NT = _EW // _K


def _proj_body(x_ref, w_ref, al_ref, ar_ref, feat_ref, el_ref, er_ref):
    f = jnp.dot(x_ref[...], w_ref[...], preferred_element_type=jnp.float32)
    feat_ref[...] = f
    el_ref[...] = jnp.sum(f * al_ref[...], axis=1, keepdims=True)
    er_ref[...] = jnp.sum(f * ar_ref[...], axis=1, keepdims=True)


def _gcn_body(adj0_ref, adj1_ref, fa0_ref, fa1_ref, w0_ref, b0_ref,
              w1_ref, b1_ref, l1a_ref, l1b_ref, lb1_ref, l_ref,
              s0_ref, s1_ref):
    f32 = jnp.float32

    @pl.when(pl.program_id(0) == 0)
    def _():
        s0_ref[...] = jnp.dot(fa0_ref[...], w0_ref[...],
                              preferred_element_type=f32)
        s1_ref[...] = jnp.dot(fa1_ref[...], w1_ref[...],
                              preferred_element_type=f32)

    o0 = jnp.dot(adj0_ref[...], s0_ref[...], preferred_element_type=f32)
    o1 = jnp.dot(adj1_ref[...], s1_ref[...], preferred_element_type=f32)
    h0 = jnp.tanh(o0 + b0_ref[...])
    h1 = jnp.tanh(o1 + b1_ref[...])
    l_ref[...] = jnp.tanh(
        jnp.dot(h0, l1a_ref[...], preferred_element_type=f32)
        + jnp.dot(h1, l1b_ref[...], preferred_element_type=f32)
        + lb1_ref[...])


def _sem_body(l_ref, agg_ref, es_ref, p1_ref, pb1_ref, p2_ref, out_ref):
    lmat = l_ref[...]
    dinv = 1.0 / (es_ref[0] + es_ref[1] + 1e-9)
    ff = jnp.tanh((agg_ref[0] + agg_ref[1]) * dinv)
    t0 = jnp.tanh(jnp.dot(lmat, p1_ref[...],
                          preferred_element_type=jnp.float32) + pb1_ref[...])
    t1 = jnp.tanh(jnp.dot(ff, p1_ref[...],
                          preferred_element_type=jnp.float32) + pb1_ref[...])
    w0 = jnp.sum(t0 * p2_ref[...]) / _N
    w1 = jnp.sum(t1 * p2_ref[...]) / _N
    m = jnp.maximum(w0, w1)
    e0 = jnp.exp(w0 - m)
    e1 = jnp.exp(w1 - m)
    den = e0 + e1
    out_ref[...] = (e0 / den) * lmat + (e1 / den) * ff


def _gat_sc_body(feat_hbm, sim_hbm, el_hbm, er_hbm,
                 esum_hbm, agg_hbm,
                 src_v, dst_v, el_v, er_v, ex_v, idx_v, dat_v,
                 zbe_v, zb_v,
                 gbuf0, gbuf1, sbuf0, sbuf1, ig0, ig1, is0, is1,
                 gsem0, gsem1, ssem0, ssem1,
                 esum_sh, out_sh):
    c = lax.axis_index("c")
    s = lax.axis_index("s")
    wid = c * _NS + s
    base = wid * _EW
    gbufs = (gbuf0, gbuf1)
    sbufs = (sbuf0, sbuf1)
    igs = (ig0, ig1)
    iss = (is0, is1)
    gsems = (gsem0, gsem1)
    ssems = (ssem0, ssem1)

    for i in range(_ROWS_W // 16):
        zbe_v[pl.ds(i * 16, 16)] = jnp.zeros((16,), jnp.float32)
    pltpu.sync_copy(zbe_v, esum_sh.at[pl.ds(s * _ROWS_W, _ROWS_W)])

    def zrow(r, carry):
        for v in range(8):
            zb_v[r, pl.ds(v * 16, 16)] = jnp.zeros((16,), jnp.float32)
        return carry
    lax.fori_loop(0, 16, zrow, 0)

    def zcp(t, carry):
        pltpu.sync_copy(zb_v, out_sh.at[pl.ds(s * _ROWS_W + t * 16, 16)])
        return carry
    lax.fori_loop(0, _ROWS_W // 16, zcp, 0)

    pltpu.sync_copy(sim_hbm.at[0, pl.ds(base, _EW)], src_v)
    pltpu.sync_copy(sim_hbm.at[1, pl.ds(base, _EW)], dst_v)
    pltpu.sync_copy(el_hbm, el_v)
    pltpu.sync_copy(er_hbm, er_v)

    def ch(i, carry):
        sl = pl.ds(i * 16, 16)
        elv = plsc.load_gather(el_v, [src_v[sl]])
        erv = plsc.load_gather(er_v, [dst_v[sl]])
        e = elv + erv
        e = jnp.where(e > 0.0, e, 0.2 * e)
        ex_v[sl] = jnp.exp(e)
        return carry
    lax.fori_loop(0, _EW // 16, ch, 0)

    plsc.subcore_barrier()

    def fill_idx(buf, src_ref, t):
        for i in range(_K // 16):
            buf[pl.ds(i * 16, 16)] = src_ref[pl.ds(t * _K + i * 16, 16)]

    for b in range(2):
        fill_idx(igs[b], src_v, b)
        pltpu.async_copy(feat_hbm.at[igs[b]], gbufs[b], gsems[b])

    def sc_ch(t, carry):
        for i in range(8):
            sl_l = pl.ds(i * 16, 16)
            sl_g = pl.ds(t * 128 + i * 16, 16)
            idx_v[sl_l] = dst_v[sl_g]
            dat_v[sl_l] = ex_v[sl_g]
        pltpu.sync_copy(dat_v, esum_sh.at[idx_v], add=True)
        return carry
    lax.fori_loop(0, _EW // 128, sc_ch, 0)

    def pair(p, carry):
        for b in range(2):
            t = p * 2 + b
            pltpu.make_async_copy(feat_hbm.at[igs[b]], gbufs[b],
                                  gsems[b]).wait()

            @pl.when(t >= 2)
            def _():
                pltpu.make_async_copy(sbufs[b], out_sh.at[iss[b]],
                                      ssems[b]).wait()

            def edge(j, carry2):
                av = plsc.load_gather(
                    ex_v, [jnp.full((16,), t * _K + j, jnp.int32)])
                for v in range(8):
                    sl = pl.ds(v * 16, 16)
                    sbufs[b][j, sl] = gbufs[b][j, sl] * av
                return carry2
            lax.fori_loop(0, _K, edge, 0)

            fill_idx(iss[b], dst_v, t)
            pltpu.async_copy(sbufs[b], out_sh.at[iss[b]], ssems[b], add=True)

            @pl.when(t + 2 < _NT)
            def _():
                fill_idx(igs[b], src_v, t + 2)
                pltpu.async_copy(feat_hbm.at[igs[b]], gbufs[b], gsems[b])
        return carry
    lax.fori_loop(0, _NT // 2, pair, 0)

    for b in range(2):
        pltpu.make_async_copy(sbufs[b], out_sh.at[iss[b]], ssems[b]).wait()

    plsc.subcore_barrier()
    pltpu.sync_copy(esum_sh.at[pl.ds(s * _ROWS_W, _ROWS_W)],
                    esum_hbm.at[c, pl.ds(s * _ROWS_W, _ROWS_W)])
    pltpu.sync_copy(out_sh.at[pl.ds(s * _ROWS_W, _ROWS_W)],
                    agg_hbm.at[c, pl.ds(s * _ROWS_W, _ROWS_W)])


def kernel(feature, adjM_0, adjM_1, similary, feature_attr_0, feature_attr_1,
           W_gcn0, b_gcn0, W_gcn1, b_gcn1, W_gat, attn_l, attn_r,
           P1, pb1, P2, L1, lb1):
    f32 = jnp.float32

    feat, el, er = pl.pallas_call(
        _proj_body,
        out_shape=[jax.ShapeDtypeStruct((_N, _H), f32),
                   jax.ShapeDtypeStruct((_N, 1), f32),
                   jax.ShapeDtypeStruct((_N, 1), f32)],
    )(feature, W_gat, attn_l.reshape(1, _H), attn_r.reshape(1, _H))

    el1 = el.reshape(_N)
    er1 = er.reshape(_N)

    mesh = plsc.VectorSubcoreMesh(core_axis_name="c", subcore_axis_name="s")
    sc_params = pltpu.CompilerParams(needs_layout_passes=False)
    gat_sc = pl.kernel(
        _gat_sc_body,
        out_type=[jax.ShapeDtypeStruct((_NC, _N), f32),
                  jax.ShapeDtypeStruct((_NC, _N, _H), f32)],
        mesh=mesh,
        compiler_params=sc_params,
        scratch_types=[
            pltpu.VMEM((_EW,), jnp.int32),
            pltpu.VMEM((_EW,), jnp.int32),
            pltpu.VMEM((_N,), f32),
            pltpu.VMEM((_N,), f32),
            pltpu.VMEM((_EW,), f32),
            pltpu.VMEM((128,), jnp.int32),
            pltpu.VMEM((128,), f32),
            pltpu.VMEM((_ROWS_W,), f32),
            pltpu.VMEM((16, _H), f32),
            pltpu.VMEM((_K, _H), f32),
            pltpu.VMEM((_K, _H), f32),
            pltpu.VMEM((_K, _H), f32),
            pltpu.VMEM((_K, _H), f32),
            pltpu.VMEM((_K,), jnp.int32),
            pltpu.VMEM((_K,), jnp.int32),
            pltpu.VMEM((_K,), jnp.int32),
            pltpu.VMEM((_K,), jnp.int32),
            pltpu.SemaphoreType.DMA,
            pltpu.SemaphoreType.DMA,
            pltpu.SemaphoreType.DMA,
            pltpu.SemaphoreType.DMA,
            pltpu.VMEM_SHARED((_N,), f32),
            pltpu.VMEM_SHARED((_N, _H), f32),
        ],
    )
    esum_part, agg_part = gat_sc(feat, similary, el1, er1)

    br = 256
    full = lambda shape: pl.BlockSpec(shape, lambda i: (0,) * len(shape))
    l = pl.pallas_call(
        _gcn_body,
        grid=(_N // br,),
        in_specs=[
            pl.BlockSpec((br, _N), lambda i: (i, 0)),
            pl.BlockSpec((br, _N), lambda i: (i, 0)),
            full((_N, _T)), full((_N, _T)),
            full((_T, _H)), full((1, _H)),
            full((_T, _H)), full((1, _H)),
            pl.BlockSpec((_H, _H), lambda i: (0, 0)),
            pl.BlockSpec((_H, _H), lambda i: (1, 0)),
            full((1, _H)),
        ],
        out_specs=pl.BlockSpec((br, _H), lambda i: (i, 0)),
        out_shape=jax.ShapeDtypeStruct((_N, _H), f32),
        scratch_shapes=[pltpu.VMEM((_N, _H), f32), pltpu.VMEM((_N, _H), f32)],
    )(adjM_0, adjM_1, feature_attr_0, feature_attr_1,
      W_gcn0, b_gcn0.reshape(1, _H), W_gcn1, b_gcn1.reshape(1, _H),
      L1, L1, lb1.reshape(1, _H))

    out = pl.pallas_call(
        _sem_body,
        out_shape=jax.ShapeDtypeStruct((_N, _H), f32),
    )(l, agg_part, esum_part.reshape(_NC, _N, 1),
      P1, pb1.reshape(1, _H), P2.reshape(1, _H))
    return out
